# baseline (speedup 1.0000x reference)
import functools

import jax
import jax.numpy as jnp
from jax.experimental import pallas as pl
from jax.experimental.pallas import tpu as pltpu

D = 256
G = 64
ROUNDS = 4
BLK = 2000


def _gru_kernel(x_ref, b_ref, w_in_ref, b_in_ref, w_scat_ref, b_scat_ref,
                w_mcat_ref, w_hh_ref, b_hh_ref, w_out_ref, b_out_ref,
                out_ref, acc_ref, *, grid):
    i = pl.program_id(0)
    f32 = jnp.float32
    bf16 = jnp.bfloat16

    state = jax.nn.relu(
        jnp.dot(x_ref[...].astype(bf16), w_in_ref[...],
                preferred_element_type=f32)
        + b_in_ref[...])

    for r in range(ROUNDS):
        scat = (jnp.dot(state.astype(bf16), w_scat_ref[r],
                        preferred_element_type=f32)
                + b_scat_ref[r])
        message = jax.nn.relu(scat[:, :D])
        mcat = jnp.dot(message.astype(bf16), w_mcat_ref[...],
                       preferred_element_type=f32)
        rg = jax.nn.sigmoid(mcat[:, :D] + scat[:, D:2 * D])
        zg = jax.nn.sigmoid(mcat[:, D:2 * D] + scat[:, 2 * D:])
        h = jnp.tanh(mcat[:, 2 * D:]
                     + jnp.dot((rg * state).astype(bf16), w_hh_ref[...],
                               preferred_element_type=f32)
                     + b_hh_ref[...])
        state = zg * h + (1.0 - zg) * state

    ids = b_ref[0]
    onehot = (jax.lax.broadcasted_iota(jnp.int32, (G, BLK), 0)
              == ids).astype(f32)
    gs = jnp.dot(onehot, state, preferred_element_type=f32)

    @pl.when(i == 0)
    def _init():
        acc_ref[...] = gs

    @pl.when(i > 0)
    def _accum():
        acc_ref[...] += gs

    @pl.when(i == grid - 1)
    def _finish():
        out_ref[...] = (jnp.dot(acc_ref[...], w_out_ref[...],
                                preferred_element_type=f32)
                        + b_out_ref[...])


def kernel(x, edge_index, batch, W_in, b_in, W_msg, b_msg, W_mr, W_mz, W_mh,
           W_hr, b_hr, W_hz, b_hz, W_hh, b_hh, W_out, b_out):
    del edge_index
    n = x.shape[0]
    m = W_out.shape[1]
    grid = pl.cdiv(n, BLK)
    n_pad = grid * BLK - n

    if n_pad:
        x = jnp.pad(x, ((0, n_pad), (0, 0)))
    batch32 = batch.astype(jnp.int32)
    if n_pad:
        batch32 = jnp.pad(batch32, (0, n_pad), constant_values=G)
    batch32 = batch32.reshape(grid, 1, BLK)

    w_scat = jnp.concatenate(
        [W_msg, jnp.broadcast_to(W_hr[None], (ROUNDS, D, D)),
         jnp.broadcast_to(W_hz[None], (ROUNDS, D, D))], axis=2)
    b_scat = jnp.concatenate(
        [b_msg, jnp.broadcast_to(b_hr[None], (ROUNDS, D)),
         jnp.broadcast_to(b_hz[None], (ROUNDS, D))], axis=1)
    b_scat = b_scat.reshape(ROUNDS, 1, 3 * D)
    w_mcat = jnp.concatenate([W_mr, W_mz, W_mh], axis=1)

    bf16 = jnp.bfloat16
    W_in_b = W_in.astype(bf16)
    w_scat = w_scat.astype(bf16)
    w_mcat = w_mcat.astype(bf16)
    W_hh_b = W_hh.astype(bf16)

    const = lambda *zeros: (lambda i: zeros)
    out = pl.pallas_call(
        functools.partial(_gru_kernel, grid=grid),
        grid=(grid,),
        in_specs=[
            pl.BlockSpec((BLK, D), lambda i: (i, 0)),
            pl.BlockSpec((1, 1, BLK), lambda i: (i, 0, 0)),
            pl.BlockSpec((D, D), const(0, 0)),
            pl.BlockSpec((1, D), const(0, 0)),
            pl.BlockSpec((ROUNDS, D, 3 * D), const(0, 0, 0)),
            pl.BlockSpec((ROUNDS, 1, 3 * D), const(0, 0, 0)),
            pl.BlockSpec((D, 3 * D), const(0, 0)),
            pl.BlockSpec((D, D), const(0, 0)),
            pl.BlockSpec((1, D), const(0, 0)),
            pl.BlockSpec((D, m), const(0, 0)),
            pl.BlockSpec((1, m), const(0, 0)),
        ],
        out_specs=pl.BlockSpec((G, m), const(0, 0)),
        out_shape=jax.ShapeDtypeStruct((G, m), jnp.float32),
        scratch_shapes=[pltpu.VMEM((G, D), jnp.float32)],
        compiler_params=pltpu.CompilerParams(
            dimension_semantics=("arbitrary",)),
    )(x, batch32, W_in_b, b_in.reshape(1, D), w_scat, b_scat, w_mcat,
      W_hh_b, b_hh.reshape(1, D), W_out, b_out.reshape(1, m))
    return out


# speedup vs baseline: 1.0206x; 1.0206x over previous
import functools

import jax
import jax.numpy as jnp
from jax.experimental import pallas as pl
from jax.experimental.pallas import tpu as pltpu

D = 256
G = 64
ROUNDS = 4
BLK = 5000


def _gru_kernel(x_ref, b_ref, w_in_ref, b_in_ref, w_scat_ref, b_scat_ref,
                w_mcat_ref, w_hh_ref, b_hh_ref, w_out_ref, b_out_ref,
                out_ref, acc_ref, *, grid):
    i = pl.program_id(0)
    f32 = jnp.float32

    state = jax.nn.relu(
        jnp.dot(x_ref[...], w_in_ref[...], preferred_element_type=f32)
        + b_in_ref[...])

    for r in range(ROUNDS):
        scat = (jnp.dot(state, w_scat_ref[r], preferred_element_type=f32)
                + b_scat_ref[r])
        message = jax.nn.relu(scat[:, :D])
        mcat = jnp.dot(message, w_mcat_ref[...], preferred_element_type=f32)
        rg = jax.nn.sigmoid(mcat[:, :D] + scat[:, D:2 * D])
        zg = jax.nn.sigmoid(mcat[:, D:2 * D] + scat[:, 2 * D:])
        h = jnp.tanh(mcat[:, 2 * D:]
                     + jnp.dot(rg * state, w_hh_ref[...],
                               preferred_element_type=f32)
                     + b_hh_ref[...])
        state = zg * h + (1.0 - zg) * state

    ids = b_ref[0]
    onehot = (jax.lax.broadcasted_iota(jnp.int32, (G, BLK), 0)
              == ids).astype(f32)
    gs = jnp.dot(onehot, state, preferred_element_type=f32)

    @pl.when(i == 0)
    def _init():
        acc_ref[...] = gs

    @pl.when(i > 0)
    def _accum():
        acc_ref[...] += gs

    @pl.when(i == grid - 1)
    def _finish():
        out_ref[...] = (jnp.dot(acc_ref[...], w_out_ref[...],
                                preferred_element_type=f32)
                        + b_out_ref[...])


def kernel(x, edge_index, batch, W_in, b_in, W_msg, b_msg, W_mr, W_mz, W_mh,
           W_hr, b_hr, W_hz, b_hz, W_hh, b_hh, W_out, b_out):
    del edge_index
    n = x.shape[0]
    m = W_out.shape[1]
    grid = pl.cdiv(n, BLK)
    n_pad = grid * BLK - n

    if n_pad:
        x = jnp.pad(x, ((0, n_pad), (0, 0)))
    batch32 = batch.astype(jnp.int32)
    if n_pad:
        batch32 = jnp.pad(batch32, (0, n_pad), constant_values=G)
    batch32 = batch32.reshape(grid, 1, BLK)

    w_scat = jnp.concatenate(
        [W_msg, jnp.broadcast_to(W_hr[None], (ROUNDS, D, D)),
         jnp.broadcast_to(W_hz[None], (ROUNDS, D, D))], axis=2)
    b_scat = jnp.concatenate(
        [b_msg, jnp.broadcast_to(b_hr[None], (ROUNDS, D)),
         jnp.broadcast_to(b_hz[None], (ROUNDS, D))], axis=1)
    b_scat = b_scat.reshape(ROUNDS, 1, 3 * D)
    w_mcat = jnp.concatenate([W_mr, W_mz, W_mh], axis=1)

    const = lambda *zeros: (lambda i: zeros)
    out = pl.pallas_call(
        functools.partial(_gru_kernel, grid=grid),
        grid=(grid,),
        in_specs=[
            pl.BlockSpec((BLK, D), lambda i: (i, 0)),
            pl.BlockSpec((1, 1, BLK), lambda i: (i, 0, 0)),
            pl.BlockSpec((D, D), const(0, 0)),
            pl.BlockSpec((1, D), const(0, 0)),
            pl.BlockSpec((ROUNDS, D, 3 * D), const(0, 0, 0)),
            pl.BlockSpec((ROUNDS, 1, 3 * D), const(0, 0, 0)),
            pl.BlockSpec((D, 3 * D), const(0, 0)),
            pl.BlockSpec((D, D), const(0, 0)),
            pl.BlockSpec((1, D), const(0, 0)),
            pl.BlockSpec((D, m), const(0, 0)),
            pl.BlockSpec((1, m), const(0, 0)),
        ],
        out_specs=pl.BlockSpec((G, m), const(0, 0)),
        out_shape=jax.ShapeDtypeStruct((G, m), jnp.float32),
        scratch_shapes=[pltpu.VMEM((G, D), jnp.float32)],
        compiler_params=pltpu.CompilerParams(
            dimension_semantics=("arbitrary",)),
    )(x, batch32, W_in, b_in.reshape(1, D), w_scat, b_scat, w_mcat,
      W_hh, b_hh.reshape(1, D), W_out, b_out.reshape(1, m))
    return out


# speedup vs baseline: 1.0917x; 1.0697x over previous
import functools

import jax
import jax.numpy as jnp
from jax.experimental import pallas as pl
from jax.experimental.pallas import tpu as pltpu

D = 256
G = 64
ROUNDS = 4
BLK = 5000


def _gru_kernel(x_ref, b_ref, w_in_ref, b_in_ref, w_msg_ref, b_msg_ref,
                w_mr_ref, w_mz_ref, w_mh_ref, w_hr_ref, b_hr_ref,
                w_hz_ref, b_hz_ref, w_hh_ref, b_hh_ref, w_out_ref, b_out_ref,
                out_ref, acc_ref, w_scat_ref, w_mcat_ref, *, grid):
    i = pl.program_id(0)
    f32 = jnp.float32

    @pl.when(i == 0)
    def _pack():
        for r in range(ROUNDS):
            w_scat_ref[r, :, :D] = w_msg_ref[r]
            w_scat_ref[r, :, D:2 * D] = w_hr_ref[...]
            w_scat_ref[r, :, 2 * D:] = w_hz_ref[...]
        w_mcat_ref[:, :D] = w_mr_ref[...]
        w_mcat_ref[:, D:2 * D] = w_mz_ref[...]
        w_mcat_ref[:, 2 * D:] = w_mh_ref[...]

    state = jax.nn.relu(
        jnp.dot(x_ref[...], w_in_ref[...], preferred_element_type=f32)
        + b_in_ref[...])

    for r in range(ROUNDS):
        scat = jnp.dot(state, w_scat_ref[r], preferred_element_type=f32)
        message = jax.nn.relu(scat[:, :D] + b_msg_ref[r])
        mcat = jnp.dot(message, w_mcat_ref[...], preferred_element_type=f32)
        rg = jax.nn.sigmoid(mcat[:, :D] + scat[:, D:2 * D] + b_hr_ref[...])
        zg = jax.nn.sigmoid(mcat[:, D:2 * D] + scat[:, 2 * D:]
                            + b_hz_ref[...])
        h = jnp.tanh(mcat[:, 2 * D:]
                     + jnp.dot(rg * state, w_hh_ref[...],
                               preferred_element_type=f32)
                     + b_hh_ref[...])
        state = zg * h + (1.0 - zg) * state

    ids = b_ref[0]
    onehot = (jax.lax.broadcasted_iota(jnp.int32, (G, BLK), 0)
              == ids).astype(f32)
    gs = jnp.dot(onehot, state, preferred_element_type=f32)

    @pl.when(i == 0)
    def _init():
        acc_ref[...] = gs

    @pl.when(i > 0)
    def _accum():
        acc_ref[...] += gs

    @pl.when(i == grid - 1)
    def _finish():
        out_ref[...] = (jnp.dot(acc_ref[...], w_out_ref[...],
                                preferred_element_type=f32)
                        + b_out_ref[...])


def kernel(x, edge_index, batch, W_in, b_in, W_msg, b_msg, W_mr, W_mz, W_mh,
           W_hr, b_hr, W_hz, b_hz, W_hh, b_hh, W_out, b_out):
    del edge_index
    n = x.shape[0]
    m = W_out.shape[1]
    grid = pl.cdiv(n, BLK)
    n_pad = grid * BLK - n

    if n_pad:
        x = jnp.pad(x, ((0, n_pad), (0, 0)))
    batch32 = batch.astype(jnp.int32)
    if n_pad:
        batch32 = jnp.pad(batch32, (0, n_pad), constant_values=G)
    batch32 = batch32.reshape(grid, 1, BLK)

    const = lambda *zeros: (lambda i: zeros)
    out = pl.pallas_call(
        functools.partial(_gru_kernel, grid=grid),
        grid=(grid,),
        in_specs=[
            pl.BlockSpec((BLK, D), lambda i: (i, 0)),
            pl.BlockSpec((1, 1, BLK), lambda i: (i, 0, 0)),
            pl.BlockSpec((D, D), const(0, 0)),
            pl.BlockSpec((1, D), const(0, 0)),
            pl.BlockSpec((ROUNDS, D, D), const(0, 0, 0)),
            pl.BlockSpec((ROUNDS, 1, D), const(0, 0, 0)),
            pl.BlockSpec((D, D), const(0, 0)),
            pl.BlockSpec((D, D), const(0, 0)),
            pl.BlockSpec((D, D), const(0, 0)),
            pl.BlockSpec((D, D), const(0, 0)),
            pl.BlockSpec((1, D), const(0, 0)),
            pl.BlockSpec((D, D), const(0, 0)),
            pl.BlockSpec((1, D), const(0, 0)),
            pl.BlockSpec((D, D), const(0, 0)),
            pl.BlockSpec((1, D), const(0, 0)),
            pl.BlockSpec((D, m), const(0, 0)),
            pl.BlockSpec((1, m), const(0, 0)),
        ],
        out_specs=pl.BlockSpec((G, m), const(0, 0)),
        out_shape=jax.ShapeDtypeStruct((G, m), jnp.float32),
        scratch_shapes=[pltpu.VMEM((G, D), jnp.float32),
                        pltpu.VMEM((ROUNDS, D, 3 * D), jnp.float32),
                        pltpu.VMEM((D, 3 * D), jnp.float32)],
        compiler_params=pltpu.CompilerParams(
            dimension_semantics=("arbitrary",)),
    )(x, batch32, W_in, b_in.reshape(1, D), W_msg,
      b_msg.reshape(ROUNDS, 1, D), W_mr, W_mz, W_mh, W_hr,
      b_hr.reshape(1, D), W_hz, b_hz.reshape(1, D), W_hh,
      b_hh.reshape(1, D), W_out, b_out.reshape(1, m))
    return out


# speedup vs baseline: 1.1578x; 1.0605x over previous
import functools

import jax
import jax.numpy as jnp
from jax.experimental import pallas as pl
from jax.experimental.pallas import tpu as pltpu

D = 256
G = 64
ROUNDS = 4
BLK = 5000


def _gru_kernel(x_ref, b_ref, w_in_ref, b_in_ref, w_msg_ref, b_msg_ref,
                w_mr_ref, w_mz_ref, w_mh_ref, w_hr_ref, b_hr_ref,
                w_hz_ref, b_hz_ref, w_hh_ref, b_hh_ref, w_out_ref, b_out_ref,
                out_ref, acc_ref, w_scat_ref, w_mcat_ref, *, grid):
    i = pl.program_id(0)
    f32 = jnp.float32

    @pl.when(i == 0)
    def _pack():
        for r in range(ROUNDS):
            w_scat_ref[r, :, :D] = w_msg_ref[r]
            w_scat_ref[r, :, D:2 * D] = w_hr_ref[...]
            w_scat_ref[r, :, 2 * D:] = w_hz_ref[...]
        w_mcat_ref[:, :D] = w_mr_ref[...]
        w_mcat_ref[:, D:2 * D] = w_mz_ref[...]
        w_mcat_ref[:, 2 * D:] = w_mh_ref[...]

    state = jax.nn.relu(
        jnp.dot(x_ref[...], w_in_ref[...], preferred_element_type=f32)
        + b_in_ref[...])

    for r in range(ROUNDS):
        scat = jnp.dot(state, w_scat_ref[r], preferred_element_type=f32)
        message = jax.nn.relu(scat[:, :D] + b_msg_ref[r])
        mcat = jnp.dot(message, w_mcat_ref[...], preferred_element_type=f32)
        rg = 0.5 + 0.5 * jnp.tanh(
            0.5 * (mcat[:, :D] + scat[:, D:2 * D] + b_hr_ref[...]))
        zg = 0.5 + 0.5 * jnp.tanh(
            0.5 * (mcat[:, D:2 * D] + scat[:, 2 * D:] + b_hz_ref[...]))
        h = jnp.tanh(mcat[:, 2 * D:]
                     + jnp.dot(rg * state, w_hh_ref[...],
                               preferred_element_type=f32)
                     + b_hh_ref[...])
        state = state + zg * (h - state)

    ids = b_ref[0]
    onehot = (jax.lax.broadcasted_iota(jnp.int32, (G, BLK), 0)
              == ids).astype(f32)
    gs = jnp.dot(onehot, state, preferred_element_type=f32)

    @pl.when(i == 0)
    def _init():
        acc_ref[...] = gs

    @pl.when(i > 0)
    def _accum():
        acc_ref[...] += gs

    @pl.when(i == grid - 1)
    def _finish():
        out_ref[...] = (jnp.dot(acc_ref[...], w_out_ref[...],
                                preferred_element_type=f32)
                        + b_out_ref[...])


def kernel(x, edge_index, batch, W_in, b_in, W_msg, b_msg, W_mr, W_mz, W_mh,
           W_hr, b_hr, W_hz, b_hz, W_hh, b_hh, W_out, b_out):
    del edge_index
    n = x.shape[0]
    m = W_out.shape[1]
    grid = pl.cdiv(n, BLK)
    n_pad = grid * BLK - n

    if n_pad:
        x = jnp.pad(x, ((0, n_pad), (0, 0)))
    batch32 = batch.astype(jnp.int32)
    if n_pad:
        batch32 = jnp.pad(batch32, (0, n_pad), constant_values=G)
    batch32 = batch32.reshape(grid, 1, BLK)

    const = lambda *zeros: (lambda i: zeros)
    out = pl.pallas_call(
        functools.partial(_gru_kernel, grid=grid),
        grid=(grid,),
        in_specs=[
            pl.BlockSpec((BLK, D), lambda i: (i, 0)),
            pl.BlockSpec((1, 1, BLK), lambda i: (i, 0, 0)),
            pl.BlockSpec((D, D), const(0, 0)),
            pl.BlockSpec((1, D), const(0, 0)),
            pl.BlockSpec((ROUNDS, D, D), const(0, 0, 0)),
            pl.BlockSpec((ROUNDS, 1, D), const(0, 0, 0)),
            pl.BlockSpec((D, D), const(0, 0)),
            pl.BlockSpec((D, D), const(0, 0)),
            pl.BlockSpec((D, D), const(0, 0)),
            pl.BlockSpec((D, D), const(0, 0)),
            pl.BlockSpec((1, D), const(0, 0)),
            pl.BlockSpec((D, D), const(0, 0)),
            pl.BlockSpec((1, D), const(0, 0)),
            pl.BlockSpec((D, D), const(0, 0)),
            pl.BlockSpec((1, D), const(0, 0)),
            pl.BlockSpec((D, m), const(0, 0)),
            pl.BlockSpec((1, m), const(0, 0)),
        ],
        out_specs=pl.BlockSpec((G, m), const(0, 0)),
        out_shape=jax.ShapeDtypeStruct((G, m), jnp.float32),
        scratch_shapes=[pltpu.VMEM((G, D), jnp.float32),
                        pltpu.VMEM((ROUNDS, D, 3 * D), jnp.float32),
                        pltpu.VMEM((D, 3 * D), jnp.float32)],
        compiler_params=pltpu.CompilerParams(
            dimension_semantics=("arbitrary",)),
    )(x, batch32, W_in, b_in.reshape(1, D), W_msg,
      b_msg.reshape(ROUNDS, 1, D), W_mr, W_mz, W_mh, W_hr,
      b_hr.reshape(1, D), W_hz, b_hz.reshape(1, D), W_hh,
      b_hh.reshape(1, D), W_out, b_out.reshape(1, m))
    return out
